# Initial kernel scaffold; baseline (speedup 1.0000x reference)
import jax
import jax.numpy as jnp
from jax.experimental import pallas as pl


def kernel(x, edge_index, W_g, b_g, W1, b1, W2, b2, W3, b3):
    raise NotImplementedError("write your pallas kernel here")


# speedup vs baseline: 35.2753x; 35.2753x over previous
import functools

import jax
import jax.numpy as jnp
from jax import lax
from jax.experimental import pallas as pl
from jax.experimental.pallas import tpu as pltpu
from jax.experimental.pallas import tpu_sc as plsc

_NW = 32
_NC = 2
_NS = 16
_C = 128


def _deg_kernel(n_s, k_blk):
    mesh = plsc.VectorSubcoreMesh(core_axis_name="c", subcore_axis_name="s")
    rows_per_tile = n_s // _NS

    @functools.partial(
        pl.kernel,
        mesh=mesh,
        compiler_params=pltpu.CompilerParams(use_tc_tiling_on_sc=False),
        out_type=jax.ShapeDtypeStruct((_NC, n_s, 16), jnp.float32),
        scratch_types=[
            pltpu.VMEM((k_blk, _C), jnp.int32),
            pltpu.VMEM((_C, 16), jnp.float32),
            pltpu.VMEM((rows_per_tile, 16), jnp.float32),
            pltpu.VMEM_SHARED((n_s, 16), jnp.float32),
        ],
    )
    def k(dst_hbm, out_hbm, idx_v, ones_v, zbuf, deg_sh):
        c = lax.axis_index("c")
        s = lax.axis_index("s")
        wid = s * _NC + c
        pltpu.sync_copy(dst_hbm.at[wid], idx_v)
        z16 = jnp.zeros((16,), jnp.float32)
        o16 = jnp.ones((16,), jnp.float32)

        def fill_body(i, carry):
            ones_v[i, :] = o16
            return carry

        lax.fori_loop(0, _C, fill_body, 0)

        def zero_body(i, carry):
            zbuf[i, :] = z16
            return carry

        lax.fori_loop(0, rows_per_tile, zero_body, 0)
        pltpu.sync_copy(zbuf, deg_sh.at[pl.ds(s * rows_per_tile, rows_per_tile)])
        plsc.subcore_barrier()

        def body(j, carry):
            pltpu.sync_copy(ones_v, deg_sh.at[idx_v.at[j]], add=True)
            return carry

        lax.fori_loop(0, k_blk, body, 0)
        plsc.subcore_barrier()
        pltpu.sync_copy(
            deg_sh.at[pl.ds(s * rows_per_tile, rows_per_tile)],
            out_hbm.at[c, pl.ds(s * rows_per_tile, rows_per_tile), :],
        )

    return k


def _scatter_kernel(n, n_s, h, k_blk):
    mesh = plsc.VectorSubcoreMesh(core_axis_name="c", subcore_axis_name="s")
    rows_per_tile = n_s // _NS

    @functools.partial(
        pl.kernel,
        mesh=mesh,
        compiler_params=pltpu.CompilerParams(use_tc_tiling_on_sc=False),
        out_type=jax.ShapeDtypeStruct((_NC, n_s, h), jnp.float32),
        scratch_types=[
            pltpu.VMEM((k_blk, _C), jnp.int32),
            pltpu.VMEM((k_blk, _C), jnp.int32),
            pltpu.VMEM((_C, h), jnp.float32),
            pltpu.VMEM((_C, h), jnp.float32),
            pltpu.VMEM((rows_per_tile, h), jnp.float32),
            pltpu.VMEM_SHARED((n_s, h), jnp.float32),
            pltpu.SemaphoreType.DMA,
            pltpu.SemaphoreType.DMA,
        ],
    )
    def k(m_hbm, src_hbm, dst_hbm, out_hbm,
          src_v, dst_v, buf_a, buf_b, zbuf, s_sh, sem_a, sem_b):
        c = lax.axis_index("c")
        s = lax.axis_index("s")
        wid = s * _NC + c
        pltpu.sync_copy(src_hbm.at[wid], src_v)
        pltpu.sync_copy(dst_hbm.at[wid], dst_v)

        z16 = jnp.zeros((16,), jnp.float32)

        def zero_body(i, carry):
            for lo in range(0, h, 16):
                zbuf[i, pl.ds(lo, 16)] = z16
            return carry

        lax.fori_loop(0, rows_per_tile, zero_body, 0)
        pltpu.sync_copy(zbuf, s_sh.at[pl.ds(s * rows_per_tile, rows_per_tile)])
        plsc.subcore_barrier()

        def fire(j, buf, sem):
            pltpu.make_async_copy(m_hbm.at[src_v.at[j]], buf, sem).start()

        def drain(j, buf, sem):
            pltpu.make_async_copy(m_hbm.at[src_v.at[j]], buf, sem).wait()

        def scat(j, buf):
            pltpu.sync_copy(buf, s_sh.at[dst_v.at[j]], add=True)

        fire(0, buf_a, sem_a)

        def body(jj, carry):
            j0 = jj * 2
            drain(j0, buf_a, sem_a)
            fire(j0 + 1, buf_b, sem_b)
            scat(j0, buf_a)
            drain(j0 + 1, buf_b, sem_b)

            @pl.when(jj < k_blk // 2 - 1)
            def _():
                fire(j0 + 2, buf_a, sem_a)

            scat(j0 + 1, buf_b)
            return carry

        lax.fori_loop(0, k_blk // 2, body, 0)
        plsc.subcore_barrier()
        pltpu.sync_copy(
            s_sh.at[pl.ds(s * rows_per_tile, rows_per_tile)],
            out_hbm.at[c, pl.ds(s * rows_per_tile, rows_per_tile), :],
        )

    return k


def _tc1_body(n, x_ref, wg_ref, w1_ref, degp_ref, m_ref, dis_ref):
    deg = 1.0 + degp_ref[0, :, 0] + degp_ref[1, :, 0]
    y = lax.rsqrt(deg)
    dis_full = y * (1.5 - 0.5 * deg * y * y)
    dis = dis_full[:n]
    wc = jnp.dot(wg_ref[...], w1_ref[...], preferred_element_type=jnp.float32)
    hc = jnp.dot(x_ref[...], wc, preferred_element_type=jnp.float32)
    m_ref[...] = hc * dis[:, None]
    dis_ref[...] = dis[:, None]


def _tc2_body(n, s_ref, m_ref, dis_ref, bg_ref, w1_ref, b1_ref,
              w2_ref, b2_ref, w3_ref, b3_ref, out_ref):
    s_sum = s_ref[0, :n, :] + s_ref[1, :n, :]
    agg = dis_ref[...] * (s_sum + m_ref[...])
    bc = jnp.dot(bg_ref[...], w1_ref[...],
                 preferred_element_type=jnp.float32) + b1_ref[...]
    a = jnp.maximum(agg + bc, 0.0)
    a = jnp.maximum(
        jnp.dot(a, w2_ref[...], preferred_element_type=jnp.float32)
        + b2_ref[...], 0.0)
    out_ref[...] = (jnp.dot(a, w3_ref[...], preferred_element_type=jnp.float32)
                    + b3_ref[...])


def kernel(x, edge_index, W_g, b_g, W1, b1, W2, b2, W3, b3):
    n, d = x.shape
    e = edge_index.shape[1]
    h = W1.shape[1]

    k_blk = -(-e // (_NW * _C))
    k_blk += k_blk % 2
    e_pad = _NW * _C * k_blk
    src = edge_index[0]
    dst = edge_index[1]
    if e_pad > e:
        pad = e_pad - e
        src = jnp.concatenate([src, jnp.zeros((pad,), src.dtype)])
        dst = jnp.concatenate([dst, jnp.full((pad,), n, dst.dtype)])
    n_s = ((n + 1 + 127) // 128) * 128

    src_g = src.reshape(_NW, k_blk, _C)
    dst_g = dst.reshape(_NW, k_blk, _C)

    degp = _deg_kernel(n_s, k_blk)(dst_g)

    m, dis = pl.pallas_call(
        functools.partial(_tc1_body, n),
        out_shape=[
            jax.ShapeDtypeStruct((n, h), jnp.float32),
            jax.ShapeDtypeStruct((n, 1), jnp.float32),
        ],
    )(x, W_g, W1, degp)

    s_part = _scatter_kernel(n, n_s, h, k_blk)(m, src_g, dst_g)

    actions = pl.pallas_call(
        functools.partial(_tc2_body, n),
        out_shape=jax.ShapeDtypeStruct((n, 1), jnp.float32),
    )(s_part, m, dis, b_g.reshape(1, d), W1, b1.reshape(1, h),
      W2, b2.reshape(1, h), W3, b3.reshape(1, 1))
    return actions


# speedup vs baseline: 39.0694x; 1.1076x over previous
import functools

import jax
import jax.numpy as jnp
from jax import lax
from jax.experimental import pallas as pl
from jax.experimental.pallas import tpu as pltpu
from jax.experimental.pallas import tpu_sc as plsc

_NW = 32
_NC = 2
_NS = 16
_C = 128
_NBUF = 8


def _deg_kernel(n_s, k_blk):
    mesh = plsc.VectorSubcoreMesh(core_axis_name="c", subcore_axis_name="s")
    rows_per_tile = n_s // _NS

    @functools.partial(
        pl.kernel,
        mesh=mesh,
        compiler_params=pltpu.CompilerParams(use_tc_tiling_on_sc=False),
        out_type=jax.ShapeDtypeStruct((_NC, n_s, 16), jnp.float32),
        scratch_types=[
            pltpu.VMEM((k_blk, _C), jnp.int32),
            pltpu.VMEM((_C, 16), jnp.float32),
            pltpu.VMEM((rows_per_tile, 16), jnp.float32),
            pltpu.VMEM_SHARED((n_s, 16), jnp.float32),
            pltpu.SemaphoreType.DMA,
        ],
    )
    def k(dst_hbm, out_hbm, idx_v, ones_v, zbuf, deg_sh, sem):
        c = lax.axis_index("c")
        s = lax.axis_index("s")
        wid = s * _NC + c
        pltpu.sync_copy(dst_hbm.at[wid], idx_v)
        z16 = jnp.zeros((16,), jnp.float32)
        o16 = jnp.ones((16,), jnp.float32)

        def fill_body(i, carry):
            ones_v[i, :] = o16
            return carry

        lax.fori_loop(0, _C, fill_body, 0)

        def zero_body(i, carry):
            zbuf[i, :] = z16
            return carry

        lax.fori_loop(0, rows_per_tile, zero_body, 0)
        pltpu.sync_copy(zbuf, deg_sh.at[pl.ds(s * rows_per_tile, rows_per_tile)])
        plsc.subcore_barrier()

        def body(j, carry):
            pltpu.async_copy(ones_v, deg_sh.at[idx_v.at[j]], sem, add=True)
            return carry

        lax.fori_loop(0, k_blk, body, 0)

        def drain_body(j, carry):
            pltpu.make_async_copy(ones_v, deg_sh.at[idx_v.at[j]], sem).wait()
            return carry

        lax.fori_loop(0, k_blk, drain_body, 0)
        plsc.subcore_barrier()
        pltpu.sync_copy(
            deg_sh.at[pl.ds(s * rows_per_tile, rows_per_tile)],
            out_hbm.at[c, pl.ds(s * rows_per_tile, rows_per_tile), :],
        )

    return k


def _scatter_kernel(n, n_s, h, k_blk):
    mesh = plsc.VectorSubcoreMesh(core_axis_name="c", subcore_axis_name="s")
    rows_per_tile = n_s // _NS

    @functools.partial(
        pl.kernel,
        mesh=mesh,
        compiler_params=pltpu.CompilerParams(use_tc_tiling_on_sc=False),
        out_type=jax.ShapeDtypeStruct((_NC, n_s, h), jnp.float32),
        scratch_types=[
            pltpu.VMEM((k_blk, _C), jnp.int32),
            pltpu.VMEM((k_blk, _C), jnp.int32),
            [pltpu.VMEM((_C, h), jnp.float32) for _ in range(_NBUF)],
            pltpu.VMEM((rows_per_tile, h), jnp.float32),
            pltpu.VMEM_SHARED((n_s, h), jnp.float32),
            [pltpu.SemaphoreType.DMA for _ in range(_NBUF)],
            [pltpu.SemaphoreType.DMA for _ in range(_NBUF)],
        ],
    )
    def k(m_hbm, src_hbm, dst_hbm, out_hbm,
          src_v, dst_v, bufs, zbuf, s_sh, gsems, ssems):
        c = lax.axis_index("c")
        s = lax.axis_index("s")
        wid = s * _NC + c
        pltpu.sync_copy(src_hbm.at[wid], src_v)
        pltpu.sync_copy(dst_hbm.at[wid], dst_v)

        z16 = jnp.zeros((16,), jnp.float32)

        def zero_body(i, carry):
            for lo in range(0, h, 16):
                zbuf[i, pl.ds(lo, 16)] = z16
            return carry

        lax.fori_loop(0, rows_per_tile, zero_body, 0)
        pltpu.sync_copy(zbuf, s_sh.at[pl.ds(s * rows_per_tile, rows_per_tile)])
        plsc.subcore_barrier()

        def g_desc(j, b):
            return pltpu.make_async_copy(m_hbm.at[src_v.at[j]], bufs[b],
                                         gsems[b])

        for b in range(_NBUF):
            g_desc(b, b).start()

        n_rounds = k_blk // _NBUF

        def body(jj, carry):
            j0 = jj * _NBUF
            scats = []
            for b in range(_NBUF):
                g_desc(j0 + b, b).wait()
                scats.append(pltpu.async_copy(
                    bufs[b], s_sh.at[dst_v.at[j0 + b]], ssems[b], add=True))
            for b in range(_NBUF):
                scats[b].wait()

                @pl.when(jj < n_rounds - 1)
                def _(b=b):
                    g_desc(j0 + b + _NBUF, b).start()

            return carry

        lax.fori_loop(0, n_rounds, body, 0)
        plsc.subcore_barrier()
        pltpu.sync_copy(
            s_sh.at[pl.ds(s * rows_per_tile, rows_per_tile)],
            out_hbm.at[c, pl.ds(s * rows_per_tile, rows_per_tile), :],
        )

    return k


def _tc1_body(n, x_ref, wg_ref, w1_ref, degp_ref, m_ref, dis_ref):
    deg = 1.0 + degp_ref[0, :, 0] + degp_ref[1, :, 0]
    y = lax.rsqrt(deg)
    dis_full = y * (1.5 - 0.5 * deg * y * y)
    dis = dis_full[:n]
    wc = jnp.dot(wg_ref[...], w1_ref[...], preferred_element_type=jnp.float32)
    hc = jnp.dot(x_ref[...], wc, preferred_element_type=jnp.float32)
    m_ref[...] = hc * dis[:, None]
    dis_ref[...] = dis[:, None]


def _tc2_body(n, s_ref, m_ref, dis_ref, bg_ref, w1_ref, b1_ref,
              w2_ref, b2_ref, w3_ref, b3_ref, out_ref):
    s_sum = s_ref[0, :n, :] + s_ref[1, :n, :]
    agg = dis_ref[...] * (s_sum + m_ref[...])
    bc = jnp.dot(bg_ref[...], w1_ref[...],
                 preferred_element_type=jnp.float32) + b1_ref[...]
    a = jnp.maximum(agg + bc, 0.0)
    a = jnp.maximum(
        jnp.dot(a, w2_ref[...], preferred_element_type=jnp.float32)
        + b2_ref[...], 0.0)
    out_ref[...] = (jnp.dot(a, w3_ref[...], preferred_element_type=jnp.float32)
                    + b3_ref[...])


def kernel(x, edge_index, W_g, b_g, W1, b1, W2, b2, W3, b3):
    n, d = x.shape
    e = edge_index.shape[1]
    h = W1.shape[1]

    k_blk = -(-e // (_NW * _C))
    k_blk = ((k_blk + _NBUF - 1) // _NBUF) * _NBUF
    e_pad = _NW * _C * k_blk
    src = edge_index[0]
    dst = edge_index[1]
    if e_pad > e:
        pad = e_pad - e
        src = jnp.concatenate([src, jnp.zeros((pad,), src.dtype)])
        dst = jnp.concatenate([dst, jnp.full((pad,), n, dst.dtype)])
    n_s = ((n + 1 + 127) // 128) * 128

    src_g = src.reshape(_NW, k_blk, _C)
    dst_g = dst.reshape(_NW, k_blk, _C)

    degp = _deg_kernel(n_s, k_blk)(dst_g)

    m, dis = pl.pallas_call(
        functools.partial(_tc1_body, n),
        out_shape=[
            jax.ShapeDtypeStruct((n, h), jnp.float32),
            jax.ShapeDtypeStruct((n, 1), jnp.float32),
        ],
    )(x, W_g, W1, degp)

    s_part = _scatter_kernel(n, n_s, h, k_blk)(m, src_g, dst_g)

    actions = pl.pallas_call(
        functools.partial(_tc2_body, n),
        out_shape=jax.ShapeDtypeStruct((n, 1), jnp.float32),
    )(s_part, m, dis, b_g.reshape(1, d), W1, b1.reshape(1, h),
      W2, b2.reshape(1, h), W3, b3.reshape(1, 1))
    return actions


# speedup vs baseline: 54.2253x; 1.3879x over previous
import functools

import jax
import jax.numpy as jnp
from jax import lax
from jax.experimental import pallas as pl
from jax.experimental.pallas import tpu as pltpu
from jax.experimental.pallas import tpu_sc as plsc

_NW = 32
_NC = 2
_NS = 16
_C = 128
_NBUF = 8


def _deg_kernel(n_s, k_blk):
    mesh = plsc.VectorSubcoreMesh(core_axis_name="c", subcore_axis_name="s")
    rows_per_tile = n_s // _NS

    @functools.partial(
        pl.kernel,
        mesh=mesh,
        compiler_params=pltpu.CompilerParams(use_tc_tiling_on_sc=False),
        out_type=jax.ShapeDtypeStruct((_NC, n_s, 16), jnp.float32),
        scratch_types=[
            pltpu.VMEM((k_blk, _C), jnp.int32),
            pltpu.VMEM((_C, 16), jnp.float32),
            pltpu.VMEM((rows_per_tile, 16), jnp.float32),
            pltpu.VMEM_SHARED((n_s, 16), jnp.float32),
            pltpu.SemaphoreType.DMA,
        ],
    )
    def k(dst_hbm, out_hbm, idx_v, ones_v, zbuf, deg_sh, sem):
        c = lax.axis_index("c")
        s = lax.axis_index("s")
        wid = s * _NC + c
        pltpu.sync_copy(dst_hbm.at[wid], idx_v)
        z16 = jnp.zeros((16,), jnp.float32)
        o16 = jnp.ones((16,), jnp.float32)

        def fill_body(i, carry):
            ones_v[i, :] = o16
            return carry

        lax.fori_loop(0, _C, fill_body, 0)

        def zero_body(i, carry):
            zbuf[i, :] = z16
            return carry

        lax.fori_loop(0, rows_per_tile, zero_body, 0)
        pltpu.sync_copy(zbuf, deg_sh.at[pl.ds(s * rows_per_tile, rows_per_tile)])
        plsc.subcore_barrier()

        def body(j, carry):
            pltpu.async_copy(ones_v, deg_sh.at[idx_v.at[j]], sem, add=True)
            return carry

        lax.fori_loop(0, k_blk, body, 0)

        def drain_body(j, carry):
            pltpu.make_async_copy(ones_v, deg_sh.at[idx_v.at[j]], sem).wait()
            return carry

        lax.fori_loop(0, k_blk, drain_body, 0)
        plsc.subcore_barrier()
        pltpu.sync_copy(
            deg_sh.at[pl.ds(s * rows_per_tile, rows_per_tile)],
            out_hbm.at[c, pl.ds(s * rows_per_tile, rows_per_tile), :],
        )

    return k


def _scatter_kernel(n, n_s, h, k_blk):
    mesh = plsc.VectorSubcoreMesh(core_axis_name="c", subcore_axis_name="s")
    rows_per_tile = n_s // _NS

    @functools.partial(
        pl.kernel,
        mesh=mesh,
        compiler_params=pltpu.CompilerParams(use_tc_tiling_on_sc=False),
        out_type=jax.ShapeDtypeStruct((_NC, n_s, h), jnp.float32),
        scratch_types=[
            pltpu.VMEM((k_blk, _C), jnp.int32),
            pltpu.VMEM((k_blk, _C), jnp.int32),
            [pltpu.VMEM((_C, h), jnp.float32) for _ in range(_NBUF)],
            pltpu.VMEM((rows_per_tile, h), jnp.float32),
            pltpu.VMEM_SHARED((n_s, h), jnp.float32),
            pltpu.VMEM_SHARED((n_s, h), jnp.float32),
            [pltpu.SemaphoreType.DMA for _ in range(_NBUF)],
            [pltpu.SemaphoreType.DMA for _ in range(_NBUF)],
        ],
    )
    def k(m_hbm, src_hbm, dst_hbm, out_hbm,
          src_v, dst_v, bufs, zbuf, s_sh, m_sh, gsems, ssems):
        c = lax.axis_index("c")
        s = lax.axis_index("s")
        wid = s * _NC + c
        pltpu.sync_copy(src_hbm.at[wid], src_v)
        pltpu.sync_copy(dst_hbm.at[wid], dst_v)
        pltpu.sync_copy(
            m_hbm.at[pl.ds(s * rows_per_tile, rows_per_tile)],
            m_sh.at[pl.ds(s * rows_per_tile, rows_per_tile)],
        )

        z16 = jnp.zeros((16,), jnp.float32)

        def zero_body(i, carry):
            for lo in range(0, h, 16):
                zbuf[i, pl.ds(lo, 16)] = z16
            return carry

        lax.fori_loop(0, rows_per_tile, zero_body, 0)
        pltpu.sync_copy(zbuf, s_sh.at[pl.ds(s * rows_per_tile, rows_per_tile)])
        plsc.subcore_barrier()

        def g_desc(j, b):
            return pltpu.make_async_copy(m_sh.at[src_v.at[j]], bufs[b],
                                         gsems[b])

        for b in range(_NBUF):
            g_desc(b, b).start()

        n_rounds = k_blk // _NBUF

        def body(jj, carry):
            j0 = jj * _NBUF
            scats = []
            for b in range(_NBUF):
                g_desc(j0 + b, b).wait()
                scats.append(pltpu.async_copy(
                    bufs[b], s_sh.at[dst_v.at[j0 + b]], ssems[b], add=True))
            for b in range(_NBUF):
                scats[b].wait()

                @pl.when(jj < n_rounds - 1)
                def _(b=b):
                    g_desc(j0 + b + _NBUF, b).start()

            return carry

        lax.fori_loop(0, n_rounds, body, 0)
        plsc.subcore_barrier()
        pltpu.sync_copy(
            s_sh.at[pl.ds(s * rows_per_tile, rows_per_tile)],
            out_hbm.at[c, pl.ds(s * rows_per_tile, rows_per_tile), :],
        )

    return k


def _tc1_body(n, n_s, x_ref, wg_ref, w1_ref, degp_ref, m_ref, dis_ref):
    deg = 1.0 + degp_ref[0, :, 0] + degp_ref[1, :, 0]
    y = lax.rsqrt(deg)
    dis_full = y * (1.5 - 0.5 * deg * y * y)
    dis = dis_full[:n]
    wc = jnp.dot(wg_ref[...], w1_ref[...], preferred_element_type=jnp.float32)
    hc = jnp.dot(x_ref[...], wc, preferred_element_type=jnp.float32)
    m_ref[:n, :] = hc * dis[:, None]
    m_ref[n:, :] = jnp.zeros((n_s - n, hc.shape[1]), jnp.float32)
    dis_ref[...] = dis[:, None]


def _tc2_body(n, s_ref, m_ref, dis_ref, bg_ref, w1_ref, b1_ref,
              w2_ref, b2_ref, w3_ref, b3_ref, out_ref):
    s_sum = s_ref[0, :n, :] + s_ref[1, :n, :]
    agg = dis_ref[...] * (s_sum + m_ref[:n, :])
    bc = jnp.dot(bg_ref[...], w1_ref[...],
                 preferred_element_type=jnp.float32) + b1_ref[...]
    a = jnp.maximum(agg + bc, 0.0)
    a = jnp.maximum(
        jnp.dot(a, w2_ref[...], preferred_element_type=jnp.float32)
        + b2_ref[...], 0.0)
    out_ref[...] = (jnp.dot(a, w3_ref[...], preferred_element_type=jnp.float32)
                    + b3_ref[...])


def kernel(x, edge_index, W_g, b_g, W1, b1, W2, b2, W3, b3):
    n, d = x.shape
    e = edge_index.shape[1]
    h = W1.shape[1]

    k_blk = -(-e // (_NW * _C))
    k_blk = ((k_blk + _NBUF - 1) // _NBUF) * _NBUF
    e_pad = _NW * _C * k_blk
    src = edge_index[0]
    dst = edge_index[1]
    if e_pad > e:
        pad = e_pad - e
        src = jnp.concatenate([src, jnp.zeros((pad,), src.dtype)])
        dst = jnp.concatenate([dst, jnp.full((pad,), n, dst.dtype)])
    n_s = ((n + 1 + 127) // 128) * 128

    src_g = src.reshape(_NW, k_blk, _C)
    dst_g = dst.reshape(_NW, k_blk, _C)

    degp = _deg_kernel(n_s, k_blk)(dst_g)

    m, dis = pl.pallas_call(
        functools.partial(_tc1_body, n, n_s),
        out_shape=[
            jax.ShapeDtypeStruct((n_s, h), jnp.float32),
            jax.ShapeDtypeStruct((n, 1), jnp.float32),
        ],
    )(x, W_g, W1, degp)

    s_part = _scatter_kernel(n, n_s, h, k_blk)(m, src_g, dst_g)

    actions = pl.pallas_call(
        functools.partial(_tc2_body, n),
        out_shape=jax.ShapeDtypeStruct((n, 1), jnp.float32),
    )(s_part, m, dis, b_g.reshape(1, d), W1, b1.reshape(1, h),
      W2, b2.reshape(1, h), W3, b3.reshape(1, 1))
    return actions


# speedup vs baseline: 54.3101x; 1.0016x over previous
import functools

import jax
import jax.numpy as jnp
from jax import lax
from jax.experimental import pallas as pl
from jax.experimental.pallas import tpu as pltpu
from jax.experimental.pallas import tpu_sc as plsc

_NW = 32
_NC = 2
_NS = 16
_C = 128
_NBUF = 8


def _deg_kernel(n_s, k_blk):
    mesh = plsc.VectorSubcoreMesh(core_axis_name="c", subcore_axis_name="s")
    rows_per_tile = n_s // _NS

    @functools.partial(
        pl.kernel,
        mesh=mesh,
        compiler_params=pltpu.CompilerParams(use_tc_tiling_on_sc=False),
        out_type=jax.ShapeDtypeStruct((_NC, n_s, 16), jnp.float32),
        scratch_types=[
            pltpu.VMEM((k_blk, _C), jnp.int32),
            pltpu.VMEM((_C, 16), jnp.float32),
            pltpu.VMEM((rows_per_tile, 16), jnp.float32),
            pltpu.VMEM_SHARED((n_s, 16), jnp.float32),
            pltpu.SemaphoreType.DMA,
        ],
    )
    def k(dst_hbm, out_hbm, idx_v, ones_v, zbuf, deg_sh, sem):
        c = lax.axis_index("c")
        s = lax.axis_index("s")
        wid = s * _NC + c
        pltpu.sync_copy(dst_hbm.at[wid], idx_v)
        z16 = jnp.zeros((16,), jnp.float32)
        o16 = jnp.ones((16,), jnp.float32)

        def fill_body(i, carry):
            ones_v[i, :] = o16
            return carry

        lax.fori_loop(0, _C, fill_body, 0)

        def zero_body(i, carry):
            zbuf[i, :] = z16
            return carry

        lax.fori_loop(0, rows_per_tile, zero_body, 0)
        pltpu.sync_copy(zbuf, deg_sh.at[pl.ds(s * rows_per_tile, rows_per_tile)])
        plsc.subcore_barrier()

        def body(j, carry):
            pltpu.async_copy(ones_v, deg_sh.at[idx_v.at[j]], sem, add=True)
            return carry

        lax.fori_loop(0, k_blk, body, 0)

        def drain_body(j, carry):
            pltpu.make_async_copy(ones_v, deg_sh.at[idx_v.at[j]], sem).wait()
            return carry

        lax.fori_loop(0, k_blk, drain_body, 0)
        plsc.subcore_barrier()
        pltpu.sync_copy(
            deg_sh.at[pl.ds(s * rows_per_tile, rows_per_tile)],
            out_hbm.at[c, pl.ds(s * rows_per_tile, rows_per_tile), :],
        )

    return k


def _scatter_kernel(n, n_s, h, k_blk):
    mesh = plsc.VectorSubcoreMesh(core_axis_name="c", subcore_axis_name="s")
    rows_per_tile = n_s // _NS

    @functools.partial(
        pl.kernel,
        mesh=mesh,
        compiler_params=pltpu.CompilerParams(use_tc_tiling_on_sc=False),
        out_type=jax.ShapeDtypeStruct((_NC, n_s, h), jnp.float32),
        scratch_types=[
            pltpu.VMEM((k_blk, _C), jnp.int32),
            pltpu.VMEM((k_blk, _C), jnp.int32),
            [pltpu.VMEM((_C, h), jnp.float32) for _ in range(_NBUF)],
            pltpu.VMEM((rows_per_tile, h), jnp.float32),
            pltpu.VMEM_SHARED((n_s, h), jnp.float32),
            pltpu.VMEM_SHARED((n_s, h), jnp.float32),
            [pltpu.SemaphoreType.DMA for _ in range(_NBUF)],
            [pltpu.SemaphoreType.DMA for _ in range(_NBUF)],
        ],
    )
    def k(m_hbm, src_hbm, dst_hbm, out_hbm,
          src_v, dst_v, bufs, zbuf, s_sh, m_sh, gsems, ssems):
        c = lax.axis_index("c")
        s = lax.axis_index("s")
        wid = s * _NC + c
        pltpu.sync_copy(src_hbm.at[wid], src_v)
        pltpu.sync_copy(dst_hbm.at[wid], dst_v)
        pltpu.sync_copy(
            m_hbm.at[pl.ds(s * rows_per_tile, rows_per_tile)],
            m_sh.at[pl.ds(s * rows_per_tile, rows_per_tile)],
        )

        z16 = jnp.zeros((16,), jnp.float32)

        def zero_body(i, carry):
            for lo in range(0, h, 16):
                zbuf[i, pl.ds(lo, 16)] = z16
            return carry

        lax.fori_loop(0, rows_per_tile, zero_body, 0)
        pltpu.sync_copy(zbuf, s_sh.at[pl.ds(s * rows_per_tile, rows_per_tile)])
        plsc.subcore_barrier()

        def g_desc(j, b):
            return pltpu.make_async_copy(m_sh.at[src_v.at[j]], bufs[b],
                                         gsems[b])

        for b in range(_NBUF):
            g_desc(b, b).start()

        n_rounds = k_blk // _NBUF

        def body(jj, carry):
            j0 = jj * _NBUF
            scats = []
            for b in range(_NBUF):
                g_desc(j0 + b, b).wait()
                scats.append(pltpu.async_copy(
                    bufs[b], s_sh.at[dst_v.at[j0 + b]], ssems[b], add=True))
            for b in range(_NBUF):
                scats[b].wait()

                @pl.when(jj < n_rounds - 1)
                def _(b=b):
                    g_desc(j0 + b + _NBUF, b).start()

            return carry

        lax.fori_loop(0, n_rounds, body, 0)
        plsc.subcore_barrier()
        pltpu.sync_copy(
            s_sh.at[pl.ds(s * rows_per_tile, rows_per_tile)],
            out_hbm.at[c, pl.ds(s * rows_per_tile, rows_per_tile), :],
        )

    return k


def _tc_hc_body(x_ref, wg_ref, w1_ref, hc_ref):
    wc = jnp.dot(wg_ref[...], w1_ref[...], preferred_element_type=jnp.float32)
    hc_ref[...] = jnp.dot(x_ref[...], wc, preferred_element_type=jnp.float32)


def _tc_m_body(n, n_s, hc_ref, degp_ref, m_ref, dis_ref):
    deg = 1.0 + degp_ref[0, :, 0] + degp_ref[1, :, 0]
    y = lax.rsqrt(deg)
    dis_full = y * (1.5 - 0.5 * deg * y * y)
    dis = dis_full[:n]
    m_ref[:n, :] = hc_ref[...] * dis[:, None]
    m_ref[n:, :] = jnp.zeros((n_s - n, hc_ref.shape[1]), jnp.float32)
    dis_ref[...] = dis[:, None]


def _tc2_body(n, s_ref, m_ref, dis_ref, bg_ref, w1_ref, b1_ref,
              w2_ref, b2_ref, w3_ref, b3_ref, out_ref):
    s_sum = s_ref[0, :n, :] + s_ref[1, :n, :]
    agg = dis_ref[...] * (s_sum + m_ref[:n, :])
    bc = jnp.dot(bg_ref[...], w1_ref[...],
                 preferred_element_type=jnp.float32) + b1_ref[...]
    a = jnp.maximum(agg + bc, 0.0)
    a = jnp.maximum(
        jnp.dot(a, w2_ref[...], preferred_element_type=jnp.float32)
        + b2_ref[...], 0.0)
    out_ref[...] = (jnp.dot(a, w3_ref[...], preferred_element_type=jnp.float32)
                    + b3_ref[...])


def kernel(x, edge_index, W_g, b_g, W1, b1, W2, b2, W3, b3):
    n, d = x.shape
    e = edge_index.shape[1]
    h = W1.shape[1]

    k_blk = -(-e // (_NW * _C))
    k_blk = ((k_blk + _NBUF - 1) // _NBUF) * _NBUF
    e_pad = _NW * _C * k_blk
    src = edge_index[0]
    dst = edge_index[1]
    if e_pad > e:
        pad = e_pad - e
        src = jnp.concatenate([src, jnp.zeros((pad,), src.dtype)])
        dst = jnp.concatenate([dst, jnp.full((pad,), n, dst.dtype)])
    n_s = ((n + 1 + 127) // 128) * 128

    src_g = src.reshape(_NW, k_blk, _C)
    dst_g = dst.reshape(_NW, k_blk, _C)

    degp = _deg_kernel(n_s, k_blk)(dst_g)

    hc = pl.pallas_call(
        _tc_hc_body,
        out_shape=jax.ShapeDtypeStruct((n, h), jnp.float32),
    )(x, W_g, W1)

    m, dis = pl.pallas_call(
        functools.partial(_tc_m_body, n, n_s),
        out_shape=[
            jax.ShapeDtypeStruct((n_s, h), jnp.float32),
            jax.ShapeDtypeStruct((n, 1), jnp.float32),
        ],
    )(hc, degp)

    s_part = _scatter_kernel(n, n_s, h, k_blk)(m, src_g, dst_g)

    actions = pl.pallas_call(
        functools.partial(_tc2_body, n),
        out_shape=jax.ShapeDtypeStruct((n, 1), jnp.float32),
    )(s_part, m, dis, b_g.reshape(1, d), W1, b1.reshape(1, h),
      W2, b2.reshape(1, h), W3, b3.reshape(1, 1))
    return actions


# speedup vs baseline: 60.2513x; 1.1094x over previous
import functools

import jax
import jax.numpy as jnp
from jax import lax
from jax.experimental import pallas as pl
from jax.experimental.pallas import tpu as pltpu
from jax.experimental.pallas import tpu_sc as plsc

_NW = 32
_NC = 2
_NS = 16
_C = 128
_NBUF = 8


def _deg_kernel(n_s, k_blk, c_blk):
    mesh = plsc.VectorSubcoreMesh(core_axis_name="c", subcore_axis_name="s")
    rows_per_tile = n_s // _NS

    @functools.partial(
        pl.kernel,
        mesh=mesh,
        compiler_params=pltpu.CompilerParams(use_tc_tiling_on_sc=False),
        out_type=jax.ShapeDtypeStruct((_NC, n_s, 16), jnp.float32),
        scratch_types=[
            pltpu.VMEM((k_blk, c_blk), jnp.int32),
            pltpu.VMEM((c_blk, 16), jnp.float32),
            pltpu.VMEM((rows_per_tile, 16), jnp.float32),
            pltpu.VMEM_SHARED((n_s, 16), jnp.float32),
            pltpu.SemaphoreType.DMA,
        ],
    )
    def k(ei_hbm, out_hbm, idx_v, ones_v, zbuf, deg_sh, sem):
        c = lax.axis_index("c")
        s = lax.axis_index("s")
        wid = s * _NC + c
        pltpu.sync_copy(ei_hbm.at[1, wid], idx_v)
        z16 = jnp.zeros((16,), jnp.float32)
        o16 = jnp.ones((16,), jnp.float32)

        def fill_body(i, carry):
            ones_v[i, :] = o16
            return carry

        lax.fori_loop(0, c_blk, fill_body, 0)

        def zero_body(i, carry):
            zbuf[i, :] = z16
            return carry

        lax.fori_loop(0, rows_per_tile, zero_body, 0)
        pltpu.sync_copy(zbuf, deg_sh.at[pl.ds(s * rows_per_tile, rows_per_tile)])
        plsc.subcore_barrier()

        def body(j, carry):
            pltpu.async_copy(ones_v, deg_sh.at[idx_v.at[j]], sem, add=True)
            return carry

        lax.fori_loop(0, k_blk, body, 0)

        def drain_body(j, carry):
            pltpu.make_async_copy(ones_v, deg_sh.at[idx_v.at[j]], sem).wait()
            return carry

        lax.fori_loop(0, k_blk, drain_body, 0)
        plsc.subcore_barrier()
        pltpu.sync_copy(
            deg_sh.at[pl.ds(s * rows_per_tile, rows_per_tile)],
            out_hbm.at[c, pl.ds(s * rows_per_tile, rows_per_tile), :],
        )

    return k


def _scatter_kernel(n, n_s, h, k_blk, c_blk):
    mesh = plsc.VectorSubcoreMesh(core_axis_name="c", subcore_axis_name="s")
    rows_per_tile = n_s // _NS

    @functools.partial(
        pl.kernel,
        mesh=mesh,
        compiler_params=pltpu.CompilerParams(use_tc_tiling_on_sc=False),
        out_type=jax.ShapeDtypeStruct((_NC, n_s, h), jnp.float32),
        scratch_types=[
            pltpu.VMEM((k_blk, c_blk), jnp.int32),
            pltpu.VMEM((k_blk, c_blk), jnp.int32),
            [pltpu.VMEM((c_blk, h), jnp.float32) for _ in range(_NBUF)],
            pltpu.VMEM((rows_per_tile, h), jnp.float32),
            pltpu.VMEM_SHARED((n_s, h), jnp.float32),
            pltpu.VMEM_SHARED((n_s, h), jnp.float32),
            [pltpu.SemaphoreType.DMA for _ in range(_NBUF)],
            [pltpu.SemaphoreType.DMA for _ in range(_NBUF)],
        ],
    )
    def k(m_hbm, ei_hbm, out_hbm,
          src_v, dst_v, bufs, zbuf, s_sh, m_sh, gsems, ssems):
        c = lax.axis_index("c")
        s = lax.axis_index("s")
        wid = s * _NC + c
        pltpu.sync_copy(ei_hbm.at[0, wid], src_v)
        pltpu.sync_copy(ei_hbm.at[1, wid], dst_v)
        pltpu.sync_copy(
            m_hbm.at[pl.ds(s * rows_per_tile, rows_per_tile)],
            m_sh.at[pl.ds(s * rows_per_tile, rows_per_tile)],
        )

        z16 = jnp.zeros((16,), jnp.float32)

        def zero_body(i, carry):
            for lo in range(0, h, 16):
                zbuf[i, pl.ds(lo, 16)] = z16
            return carry

        lax.fori_loop(0, rows_per_tile, zero_body, 0)
        pltpu.sync_copy(zbuf, s_sh.at[pl.ds(s * rows_per_tile, rows_per_tile)])
        plsc.subcore_barrier()

        def g_desc(j, b):
            return pltpu.make_async_copy(m_sh.at[src_v.at[j]], bufs[b],
                                         gsems[b])

        for b in range(_NBUF):
            g_desc(b, b).start()

        n_rounds = k_blk // _NBUF

        def body(jj, carry):
            j0 = jj * _NBUF
            scats = []
            for b in range(_NBUF):
                g_desc(j0 + b, b).wait()
                scats.append(pltpu.async_copy(
                    bufs[b], s_sh.at[dst_v.at[j0 + b]], ssems[b], add=True))
            for b in range(_NBUF):
                scats[b].wait()

                @pl.when(jj < n_rounds - 1)
                def _(b=b):
                    g_desc(j0 + b + _NBUF, b).start()

            return carry

        lax.fori_loop(0, n_rounds, body, 0)
        plsc.subcore_barrier()
        pltpu.sync_copy(
            s_sh.at[pl.ds(s * rows_per_tile, rows_per_tile)],
            out_hbm.at[c, pl.ds(s * rows_per_tile, rows_per_tile), :],
        )

    return k


def _tc_hc_body(x_ref, wg_ref, w1_ref, hc_ref):
    wc = jnp.dot(wg_ref[...], w1_ref[...], preferred_element_type=jnp.float32)
    hc_ref[...] = jnp.dot(x_ref[...], wc, preferred_element_type=jnp.float32)


def _tc_m_body(n, n_s, hc_ref, degp_ref, m_ref):
    deg = 1.0 + degp_ref[0, :, 0] + degp_ref[1, :, 0]
    y = lax.rsqrt(deg)
    dis_full = y * (1.5 - 0.5 * deg * y * y)
    dis = dis_full[:n]
    m_ref[:n, :] = hc_ref[...] * dis[:, None]
    m_ref[n:, :] = jnp.zeros((n_s - n, hc_ref.shape[1]), jnp.float32)


def _tc2_body(n, s_ref, m_ref, degp_ref, bg_ref, w1_ref, b1_ref,
              w2_ref, b2_ref, w3_ref, b3_ref, out_ref):
    deg = 1.0 + degp_ref[0, :n, 0] + degp_ref[1, :n, 0]
    y = lax.rsqrt(deg)
    dis = y * (1.5 - 0.5 * deg * y * y)
    s_sum = s_ref[0, :n, :] + s_ref[1, :n, :]
    agg = dis[:, None] * (s_sum + m_ref[:n, :])
    bc = jnp.dot(bg_ref[...], w1_ref[...],
                 preferred_element_type=jnp.float32) + b1_ref[...]
    a = jnp.maximum(agg + bc, 0.0)
    a = jnp.maximum(
        jnp.dot(a, w2_ref[...], preferred_element_type=jnp.float32)
        + b2_ref[...], 0.0)
    out_ref[...] = (jnp.dot(a, w3_ref[...], preferred_element_type=jnp.float32)
                    + b3_ref[...])


def kernel(x, edge_index, W_g, b_g, W1, b1, W2, b2, W3, b3):
    n, d = x.shape
    e = edge_index.shape[1]
    h = W1.shape[1]

    c_blk = None
    if e % _NW == 0:
        e_pt = e // _NW
        for cand in range(128, 63, -1):
            if e_pt % cand == 0 and (e_pt // cand) % _NBUF == 0:
                c_blk = cand
                break
    n_s = ((n + 1 + 127) // 128) * 128
    if c_blk is not None:
        k_blk = e // (_NW * c_blk)
        ei4 = edge_index.reshape(2, _NW, k_blk, c_blk)
    else:
        c_blk = 128
        k_blk = -(-e // (_NW * c_blk))
        k_blk = ((k_blk + _NBUF - 1) // _NBUF) * _NBUF
        e_pad = _NW * c_blk * k_blk
        pad = e_pad - e
        src_p = jnp.concatenate([edge_index[0],
                                 jnp.zeros((pad,), edge_index.dtype)])
        dst_p = jnp.concatenate([edge_index[1],
                                 jnp.full((pad,), n, edge_index.dtype)])
        ei4 = jnp.stack([src_p, dst_p]).reshape(2, _NW, k_blk, c_blk)

    degp = _deg_kernel(n_s, k_blk, c_blk)(ei4)

    hc = pl.pallas_call(
        _tc_hc_body,
        out_shape=jax.ShapeDtypeStruct((n, h), jnp.float32),
    )(x, W_g, W1)

    m = pl.pallas_call(
        functools.partial(_tc_m_body, n, n_s),
        out_shape=jax.ShapeDtypeStruct((n_s, h), jnp.float32),
    )(hc, degp)

    s_part = _scatter_kernel(n, n_s, h, k_blk, c_blk)(m, ei4)

    actions = pl.pallas_call(
        functools.partial(_tc2_body, n),
        out_shape=jax.ShapeDtypeStruct((n, 1), jnp.float32),
    )(s_part, m, degp, b_g.reshape(1, d), W1, b1.reshape(1, h),
      W2, b2.reshape(1, h), W3, b3.reshape(1, 1))
    return actions


# speedup vs baseline: 61.4031x; 1.0191x over previous
import functools

import jax
import jax.numpy as jnp
from jax import lax
from jax.experimental import pallas as pl
from jax.experimental.pallas import tpu as pltpu
from jax.experimental.pallas import tpu_sc as plsc

_NW = 32
_NC = 2
_NS = 16
_C = 128
_NBUF = 8


def _deg_kernel(n_s, k_blk, c_blk):
    mesh = plsc.VectorSubcoreMesh(core_axis_name="c", subcore_axis_name="s")
    rows_per_tile = n_s // _NS

    @functools.partial(
        pl.kernel,
        mesh=mesh,
        compiler_params=pltpu.CompilerParams(use_tc_tiling_on_sc=False),
        out_type=jax.ShapeDtypeStruct((_NC, n_s, 16), jnp.float32),
        scratch_types=[
            pltpu.VMEM((k_blk, c_blk), jnp.int32),
            pltpu.VMEM((c_blk, 16), jnp.float32),
            pltpu.VMEM((rows_per_tile, 16), jnp.float32),
            pltpu.VMEM_SHARED((n_s, 16), jnp.float32),
            pltpu.SemaphoreType.DMA,
        ],
    )
    def k(ei_hbm, out_hbm, idx_v, ones_v, zbuf, deg_sh, sem):
        c = lax.axis_index("c")
        s = lax.axis_index("s")
        wid = s * _NC + c
        pltpu.sync_copy(ei_hbm.at[1, wid], idx_v)
        z16 = jnp.zeros((16,), jnp.float32)
        o16 = jnp.ones((16,), jnp.float32)

        def fill_body(i, carry):
            ones_v[i, :] = o16
            return carry

        lax.fori_loop(0, c_blk, fill_body, 0)

        def zero_body(i, carry):
            zbuf[i, :] = z16
            return carry

        lax.fori_loop(0, rows_per_tile, zero_body, 0)
        pltpu.sync_copy(zbuf, deg_sh.at[pl.ds(s * rows_per_tile, rows_per_tile)])
        plsc.subcore_barrier()

        def body(j, carry):
            pltpu.async_copy(ones_v, deg_sh.at[idx_v.at[j]], sem, add=True)
            return carry

        lax.fori_loop(0, k_blk, body, 0)

        def drain_body(j, carry):
            pltpu.make_async_copy(ones_v, deg_sh.at[idx_v.at[j]], sem).wait()
            return carry

        lax.fori_loop(0, k_blk, drain_body, 0)
        plsc.subcore_barrier()
        pltpu.sync_copy(
            deg_sh.at[pl.ds(s * rows_per_tile, rows_per_tile)],
            out_hbm.at[c, pl.ds(s * rows_per_tile, rows_per_tile), :],
        )

    return k


def _scatter_kernel(n, n_s, h, k_blk, c_blk):
    mesh = plsc.VectorSubcoreMesh(core_axis_name="c", subcore_axis_name="s")
    rows_per_tile = n_s // _NS

    @functools.partial(
        pl.kernel,
        mesh=mesh,
        compiler_params=pltpu.CompilerParams(use_tc_tiling_on_sc=False),
        out_type=jax.ShapeDtypeStruct((_NC, n_s, h), jnp.float32),
        scratch_types=[
            pltpu.VMEM((k_blk, c_blk), jnp.int32),
            pltpu.VMEM((k_blk, c_blk), jnp.int32),
            [pltpu.VMEM((c_blk, h), jnp.float32) for _ in range(_NBUF)],
            pltpu.VMEM((rows_per_tile, h), jnp.float32),
            pltpu.VMEM_SHARED((n_s, h), jnp.float32),
            pltpu.VMEM_SHARED((n_s, h), jnp.float32),
            [pltpu.SemaphoreType.DMA for _ in range(_NBUF)],
            [pltpu.SemaphoreType.DMA for _ in range(_NBUF)],
        ],
    )
    def k(m_hbm, ei_hbm, out_hbm,
          src_v, dst_v, bufs, zbuf, s_sh, m_sh, gsems, ssems):
        c = lax.axis_index("c")
        s = lax.axis_index("s")
        wid = s * _NC + c
        pltpu.sync_copy(ei_hbm.at[0, wid], src_v)
        pltpu.sync_copy(ei_hbm.at[1, wid], dst_v)
        pltpu.sync_copy(
            m_hbm.at[pl.ds(s * rows_per_tile, rows_per_tile)],
            m_sh.at[pl.ds(s * rows_per_tile, rows_per_tile)],
        )

        z16 = jnp.zeros((16,), jnp.float32)

        def zero_body(i, carry):
            for lo in range(0, h, 16):
                zbuf[i, pl.ds(lo, 16)] = z16
            return carry

        lax.fori_loop(0, rows_per_tile, zero_body, 0)
        pltpu.sync_copy(zbuf, s_sh.at[pl.ds(s * rows_per_tile, rows_per_tile)])
        plsc.subcore_barrier()

        def g_desc(j, b):
            return pltpu.make_async_copy(m_sh.at[src_v.at[j]], bufs[b],
                                         gsems[b])

        for b in range(_NBUF):
            g_desc(b, b).start()

        n_rounds = k_blk // _NBUF

        def body(jj, carry):
            j0 = jj * _NBUF
            scats = []
            for b in range(_NBUF):
                g_desc(j0 + b, b).wait()
                scats.append(pltpu.async_copy(
                    bufs[b], s_sh.at[dst_v.at[j0 + b]], ssems[b], add=True))
            for b in range(_NBUF):
                scats[b].wait()

                @pl.when(jj < n_rounds - 1)
                def _(b=b):
                    g_desc(j0 + b + _NBUF, b).start()

            return carry

        lax.fori_loop(0, n_rounds, body, 0)
        plsc.subcore_barrier()
        pltpu.sync_copy(
            s_sh.at[pl.ds(s * rows_per_tile, rows_per_tile)],
            out_hbm.at[c, pl.ds(s * rows_per_tile, rows_per_tile), :],
        )

    return k


def _tc1_body(n, n_s, x_ref, wg_ref, w1_ref, degp_ref, m_ref, dis_ref):
    deg = 1.0 + degp_ref[0, :, 0] + degp_ref[1, :, 0]
    y = lax.rsqrt(deg)
    dis_full = y * (1.5 - 0.5 * deg * y * y)
    dis = dis_full[:n]
    wc = jnp.dot(wg_ref[...], w1_ref[...], preferred_element_type=jnp.float32)
    hc = jnp.dot(x_ref[...], wc, preferred_element_type=jnp.float32)
    m_ref[:n, :] = hc * dis[:, None]
    m_ref[n:, :] = jnp.zeros((n_s - n, hc.shape[1]), jnp.float32)
    dis_ref[...] = dis_full.reshape(dis_ref.shape)


def _tc2_body(n, s_ref, m_ref, dis_ref, bg_ref, w1_ref, b1_ref,
              w2_ref, b2_ref, w3_ref, b3_ref, out_ref):
    dis = dis_ref[...].reshape(-1)[:n]
    s_sum = s_ref[0, :n, :] + s_ref[1, :n, :]
    agg = dis[:, None] * (s_sum + m_ref[:n, :])
    bc = jnp.dot(bg_ref[...], w1_ref[...],
                 preferred_element_type=jnp.float32) + b1_ref[...]
    a = jnp.maximum(agg + bc, 0.0)
    a = jnp.maximum(
        jnp.dot(a, w2_ref[...], preferred_element_type=jnp.float32)
        + b2_ref[...], 0.0)
    out_ref[...] = (jnp.dot(a, w3_ref[...], preferred_element_type=jnp.float32)
                    + b3_ref[...])


def kernel(x, edge_index, W_g, b_g, W1, b1, W2, b2, W3, b3):
    n, d = x.shape
    e = edge_index.shape[1]
    h = W1.shape[1]

    c_blk = None
    if e % _NW == 0:
        e_pt = e // _NW
        for cand in range(128, 63, -1):
            if e_pt % cand == 0 and (e_pt // cand) % _NBUF == 0:
                c_blk = cand
                break
    n_s = ((n + 1 + 127) // 128) * 128
    if c_blk is not None:
        k_blk = e // (_NW * c_blk)
        ei4 = edge_index.reshape(2, _NW, k_blk, c_blk)
    else:
        c_blk = 128
        k_blk = -(-e // (_NW * c_blk))
        k_blk = ((k_blk + _NBUF - 1) // _NBUF) * _NBUF
        e_pad = _NW * c_blk * k_blk
        pad = e_pad - e
        src_p = jnp.concatenate([edge_index[0],
                                 jnp.zeros((pad,), edge_index.dtype)])
        dst_p = jnp.concatenate([edge_index[1],
                                 jnp.full((pad,), n, edge_index.dtype)])
        ei4 = jnp.stack([src_p, dst_p]).reshape(2, _NW, k_blk, c_blk)

    degp = _deg_kernel(n_s, k_blk, c_blk)(ei4)

    m, dis2 = pl.pallas_call(
        functools.partial(_tc1_body, n, n_s),
        out_shape=[
            jax.ShapeDtypeStruct((n_s, h), jnp.float32),
            jax.ShapeDtypeStruct((n_s // 128, 128), jnp.float32),
        ],
    )(x, W_g, W1, degp)

    s_part = _scatter_kernel(n, n_s, h, k_blk, c_blk)(m, ei4)

    actions = pl.pallas_call(
        functools.partial(_tc2_body, n),
        out_shape=jax.ShapeDtypeStruct((n, 1), jnp.float32),
    )(s_part, m, dis2, b_g.reshape(1, d), W1, b1.reshape(1, h),
      W2, b2.reshape(1, h), W3, b3.reshape(1, 1))
    return actions
